# baseline (device time: 35209 ns/iter reference)
import jax
import jax.numpy as jnp
from jax import lax
from jax.experimental import pallas as pl
from jax.experimental.pallas import tpu as pltpu

B = 4
S_HALF = 256
K = 512
N = 1024
M = B * S_HALF


def kernel(O, Wo):
    o2 = O.reshape(B, 2 * S_HALF, K).astype(jnp.bfloat16)
    w2 = Wo.astype(jnp.bfloat16)

    def body(o_ref, w_ref, out_ref, send_buf, recv_buf, send_sem, recv_sem):
        my_x = lax.axis_index("x")
        my_y = lax.axis_index("y")
        peer_y = 1 - my_y

        barrier = pltpu.get_barrier_semaphore()
        pl.semaphore_signal(
            barrier, inc=1,
            device_id=(my_x, peer_y), device_id_type=pl.DeviceIdType.MESH,
        )
        pl.semaphore_wait(barrier, 1)

        w = w_ref[:, :]

        a_peer = o_ref[:, pl.ds(peer_y * S_HALF, S_HALF), :].reshape(M, K)
        send_buf[:, :] = jnp.dot(
            a_peer, w, preferred_element_type=jnp.float32
        ).astype(jnp.bfloat16)

        rdma = pltpu.make_async_remote_copy(
            src_ref=send_buf,
            dst_ref=recv_buf,
            send_sem=send_sem,
            recv_sem=recv_sem,
            device_id=(my_x, peer_y),
            device_id_type=pl.DeviceIdType.MESH,
        )
        rdma.start()

        a_mine = o_ref[:, pl.ds(my_y * S_HALF, S_HALF), :].reshape(M, K)
        out_ref[:, :] = jnp.dot(a_mine, w, preferred_element_type=jnp.float32)

        rdma.wait()
        out_ref[:, :] = out_ref[:, :] + recv_buf[:, :].astype(jnp.float32)

    out = pl.pallas_call(
        body,
        out_shape=jax.ShapeDtypeStruct((M, N), jnp.float32),
        in_specs=[
            pl.BlockSpec(memory_space=pltpu.VMEM),
            pl.BlockSpec(memory_space=pltpu.VMEM),
        ],
        out_specs=pl.BlockSpec(memory_space=pltpu.VMEM),
        scratch_shapes=[
            pltpu.VMEM((M, N), jnp.bfloat16),
            pltpu.VMEM((M, N), jnp.bfloat16),
            pltpu.SemaphoreType.DMA,
            pltpu.SemaphoreType.DMA,
        ],
        compiler_params=pltpu.CompilerParams(collective_id=0),
    )(o2, w2)
    return out.reshape(B, S_HALF, N)


# device time: 32563 ns/iter; 1.0813x vs baseline; 1.0813x over previous
import jax
import jax.numpy as jnp
from jax import lax
from jax.experimental import pallas as pl
from jax.experimental.pallas import tpu as pltpu

B = 4
S_HALF = 256
K = 512
N = 1024


def kernel(O, Wo):
    o2 = O.reshape(B, 2 * S_HALF, K)

    def body(o_ref, w_ref, out_ref, send_buf, recv_buf, send_sems, recv_sems):
        my_x = lax.axis_index("x")
        my_y = lax.axis_index("y")
        peer_y = 1 - my_y

        barrier = pltpu.get_barrier_semaphore()
        pl.semaphore_signal(
            barrier, inc=1,
            device_id=(my_x, peer_y), device_id_type=pl.DeviceIdType.MESH,
        )
        pl.semaphore_wait(barrier, 1)

        w = w_ref[:, :].astype(jnp.bfloat16)

        rdmas = []
        for c in range(B):
            a = o_ref[c, pl.ds(peer_y * S_HALF, S_HALF), :].astype(jnp.bfloat16)
            send_buf[c, :, :] = jnp.dot(
                a, w, preferred_element_type=jnp.float32
            ).astype(jnp.bfloat16)
            r = pltpu.make_async_remote_copy(
                src_ref=send_buf.at[c],
                dst_ref=recv_buf.at[c],
                send_sem=send_sems.at[c],
                recv_sem=recv_sems.at[c],
                device_id=(my_x, peer_y),
                device_id_type=pl.DeviceIdType.MESH,
            )
            r.start()
            rdmas.append(r)

        for c in range(B):
            a = o_ref[c, pl.ds(my_y * S_HALF, S_HALF), :].astype(jnp.bfloat16)
            out_ref[c, :, :] = jnp.dot(a, w, preferred_element_type=jnp.float32)

        for c in range(B):
            rdmas[c].wait_recv()
            out_ref[c, :, :] = out_ref[c, :, :] + recv_buf[c, :, :].astype(
                jnp.float32
            )
        for c in range(B):
            rdmas[c].wait_send()

    return pl.pallas_call(
        body,
        out_shape=jax.ShapeDtypeStruct((B, S_HALF, N), jnp.float32),
        in_specs=[
            pl.BlockSpec(memory_space=pltpu.VMEM),
            pl.BlockSpec(memory_space=pltpu.VMEM),
        ],
        out_specs=pl.BlockSpec(memory_space=pltpu.VMEM),
        scratch_shapes=[
            pltpu.VMEM((B, S_HALF, N), jnp.bfloat16),
            pltpu.VMEM((B, S_HALF, N), jnp.bfloat16),
            pltpu.SemaphoreType.DMA((B,)),
            pltpu.SemaphoreType.DMA((B,)),
        ],
        compiler_params=pltpu.CompilerParams(collective_id=0),
    )(o2, Wo)
